# baseline (speedup 1.0000x reference)
import functools

import jax
import jax.numpy as jnp
import numpy as np
from jax import lax
from jax.experimental import pallas as pl
from jax.experimental.pallas import tpu as pltpu
from jax.experimental.pallas import tpu_sc as plsc

EMBED_DIM = 64
NUM_EMBED = 64
BB = 128
NPIX7 = 49
NL = NPIX7 * BB

_INTERPRET = False


def _dot(a, b):
    return jax.lax.dot_general(a, b, (((1,), (0,)), ((), ())),
                               preferred_element_type=jnp.float32)


_E2 = [
    dict(r=0, s=0, feeds=[(1, 1), (2, 0), (0, 2)]),
    dict(r=1, s=0, feeds=[(2, 2), (3, 1)]),
    dict(r=1, s=-1, feeds=[(0, 0)]),
]
_E3 = [
    dict(p=0, s=0, feeds=[(1, 0), (0, 1)]),
    dict(p=1, s=0, feeds=[(2, 0), (1, 1), (0, 2)]),
    dict(p=2, s=0, feeds=[(3, 0), (2, 1), (1, 2)]),
    dict(p=3, s=0, feeds=[(3, 1), (2, 2)]),
    dict(p=0, s=1, feeds=[(3, 2)]),
    dict(p=3, s=-1, feeds=[(0, 0)]),
]


def _enc_body(x_ref, a1_ref, c1b_ref, w2f_ref, c2b_ref, enct_ref, encb_ref,
              cb_ref, z_ref, idx_ref, xp, h1p, p2, hft):
    i = pl.program_id(0)

    @pl.when(i == 0)
    def _zero():
        xp[:] = jnp.zeros((900, BB), jnp.float32)
        h1p[:] = jnp.zeros((7200, BB), jnp.float32)

    xt = x_ref[:]
    for y in range(28):
        xp[y * 30:y * 30 + 28, :] = xt[y * 28:(y + 1) * 28, :]

    a1 = a1_ref[:]
    c1b = c1b_ref[:]
    for i1 in range(14):
        rows = xp[2 * i1 * 30: 2 * i1 * 30 + 90, :]
        val = jnp.maximum(_dot(a1, rows) + c1b, 0.0)
        h1p[i1 * 480: i1 * 480 + 448, :] = val

    for ij in range(NPIX7):
        i2, j2 = divmod(ij, 7)
        for dy in range(3):
            src = ((2 * i2 + dy) * 15 + 2 * j2) * 32
            p2[dy * 96:(dy + 1) * 96, ij * BB:(ij + 1) * BB] = \
                h1p[src: src + 96, :]
    h2 = jnp.maximum(_dot(w2f_ref[:], p2[:]) + c2b_ref[:], 0.0)

    for ij in range(NPIX7):
        hft[ij * 64:(ij + 1) * 64, :] = h2[:, ij * BB:(ij + 1) * BB]
    zt = _dot(enct_ref[:], hft[:]) + encb_ref[:]
    z_ref[:] = zt

    cb = cb_ref[:]
    cbn = jnp.sum(cb * cb, axis=1, keepdims=True)
    dist = cbn - 2.0 * _dot(cb, zt)
    mn = jnp.min(dist, axis=0, keepdims=True)
    ki = jax.lax.broadcasted_iota(jnp.int32, (NUM_EMBED, BB), 0)
    cand = jnp.where(dist <= mn, ki, NUM_EMBED)
    idx = jnp.min(cand, axis=0, keepdims=True)
    idx_ref[:] = idx.reshape(1, 1, BB)


def _shift2d(src, su, sv):
    plan = []
    v0 = max(0, -sv)
    v1 = 7 - max(0, sv)
    for u in range(max(0, -su), 7 - max(0, su)):
        dst = (u * 7 + v0) * BB
        srcl = ((u + su) * 7 + v0 + sv) * BB
        plan.append((dst, srcl, (v1 - v0) * BB))
    return plan


def _dec_body(idx_ref, dct_ref, decb_ref, ws1_ref, b1_ref,
              ws2_ref, b2_ref, w3_ref, b3_ref, perm_ref,
              rec_ref, d1l, shm0, sh0m, shmm, p1ph, psh, sall3):
    i = pl.program_id(0)

    @pl.when(i == 0)
    def _zero():
        shm0[:] = jnp.zeros((32, NL), jnp.float32)
        sh0m[:] = jnp.zeros((32, NL), jnp.float32)
        shmm[:] = jnp.zeros((32, NL), jnp.float32)
        psh[:] = jnp.zeros((320, NL), jnp.float32)
        sall3[:] = jnp.zeros((1152, NL), jnp.float32)

    ki = jax.lax.broadcasted_iota(jnp.int32, (NUM_EMBED, BB), 0)
    oh = (ki == idx_ref[0]).astype(jnp.float32)
    d1t = jnp.maximum(_dot(dct_ref[:], oh) + decb_ref[:], 0.0)

    for p in range(NPIX7):
        d1l[:, p * BB:(p + 1) * BB] = d1t[p * 32:(p + 1) * 32, :]

    for dst, src, w in _shift2d(None, -1, 0):
        shm0[:, dst:dst + w] = d1l[:, src:src + w]
    for dst, src, w in _shift2d(None, 0, -1):
        sh0m[:, dst:dst + w] = d1l[:, src:src + w]
    for dst, src, w in _shift2d(None, -1, -1):
        shmm[:, dst:dst + w] = d1l[:, src:src + w]

    ws1 = ws1_ref[:]
    r00 = _dot(ws1[0:256, :], d1l[:])
    rm0 = _dot(ws1[256:384, :], shm0[:])
    r0m = _dot(ws1[384:512, :], sh0m[:])
    rmm = _dot(ws1[512:576, :], shmm[:])
    b1 = b1_ref[:]
    p1ph[0:64, :] = jnp.maximum(
        r00[0:64] + rm0[0:64] + r0m[0:64] + rmm + b1, 0.0)
    p1ph[64:128, :] = jnp.maximum(r00[64:128] + rm0[64:128] + b1, 0.0)
    p1ph[128:192, :] = jnp.maximum(r00[128:192] + r0m[64:128] + b1, 0.0)
    p1ph[192:256, :] = jnp.maximum(r00[192:256] + b1, 0.0)

    shifted = {(0, 1, 0, -1): 0, (1, 1, 0, -1): 1,
               (1, 0, -1, 0): 2, (1, 1, -1, 0): 3, (1, 1, -1, -1): 4}
    for (ry, rx, su, sv), slot in shifted.items():
        src_rows = (ry * 2 + rx) * 64
        for dst, src, w in _shift2d(None, su, sv):
            psh[slot * 64:(slot + 1) * 64, dst:dst + w] = \
                p1ph[src_rows:src_rows + 64, src:src + w]

    b2 = b2_ref[:]
    ws2 = ws2_ref[:]
    acc = {}
    off = 0
    for ey in _E2:
        for ex in _E2:
            ry, rx = ey['r'], ex['r']
            su, sv = ey['s'], ex['s']
            n = len(ey['feeds']) * len(ex['feeds'])
            if su == 0 and sv == 0:
                src = p1ph[(ry * 2 + rx) * 64:(ry * 2 + rx) * 64 + 64, :]
            else:
                slot = shifted[(ry, rx, su, sv)]
                src = psh[slot * 64:(slot + 1) * 64, :]
            res = _dot(ws2[off:off + 32 * n, :], src)
            off += 32 * n
            k = 0
            for (py, _) in ey['feeds']:
                for (px, _) in ex['feeds']:
                    acc.setdefault((py, px), []).append(
                        res[k * 32:(k + 1) * 32])
                    k += 1
    for (py, px), parts in acc.items():
        tot = parts[0]
        for q in parts[1:]:
            tot = tot + q
        slot = py * 6 + px
        sall3[slot * 32:(slot + 1) * 32, :] = jnp.maximum(tot + b2, 0.0)

    for ie in range(6):
        for je in range(6):
            if ie < 4 and je < 4:
                continue
            sy, sx = _E3[ie]['p'], _E3[je]['p']
            su, sv = _E3[ie]['s'], _E3[je]['s']
            srow = (sy * 6 + sx) * 32
            drow = (ie * 6 + je) * 32
            for dst, src, w in _shift2d(None, su, sv):
                sall3[drow:drow + 32, dst:dst + w] = \
                    sall3[srow:srow + 32, src:src + w]

    r3 = jax.nn.sigmoid(_dot(w3_ref[:], sall3[:]) + b3_ref[:])

    cat = [r3[:, g * BB:(g + 1) * BB] for g in range(NPIX7)]
    cat2 = jnp.concatenate(cat, axis=0)
    rec_ref[:] = _dot(perm_ref[:], cat2)


def _loss_body(q_ref, zt_ref, loss_ref):
    i = pl.program_id(0)

    @pl.when(i == 0)
    def _zero():
        loss_ref[:] = jnp.zeros((8, 128), jnp.float32)

    qt = q_ref[:, 0:EMBED_DIM].T
    diff = qt - zt_ref[:]
    loss_ref[:] += jnp.sum(diff * diff)


def _sc_gather(codebook, idx):
    table = jnp.pad(codebook, ((0, 0), (0, 128 - EMBED_DIM)))
    info = plsc.get_sparse_core_info()
    nw = info.num_cores * info.num_subcores
    n = idx.shape[0]
    bpw = n // nw
    mesh = plsc.VectorSubcoreMesh(core_axis_name="c", subcore_axis_name="s")

    @functools.partial(
        pl.kernel, mesh=mesh,
        out_type=jax.ShapeDtypeStruct((n, 128), jnp.float32),
        scratch_types=[
            pltpu.VMEM((bpw,), jnp.int32),
            pltpu.VMEM((bpw, 128), jnp.float32),
            pltpu.SemaphoreType.DMA,
        ],
    )
    def k(table_hbm, idx_hbm, out_hbm, idx_v, rows_v, sem):
        wid = lax.axis_index("s") * info.num_cores + lax.axis_index("c")
        base = wid * bpw
        pltpu.sync_copy(idx_hbm.at[pl.ds(base, bpw)], idx_v)
        pltpu.async_copy(table_hbm.at[idx_v], rows_v, sem).wait()
        pltpu.sync_copy(rows_v, out_hbm.at[pl.ds(base, bpw)])

    return k(table, idx)


def kernel(inputs, conv1_w, conv1_b, conv2_w, conv2_b, enc_w, enc_b, codebook,
           dec_w, dec_b, deconv1_w, deconv1_b, deconv2_w, deconv2_b,
           deconv3_w, deconv3_b):
    batch = inputs.shape[0]
    nblk = batch // BB
    f32 = jnp.float32

    w1p = conv1_w[:, :, 0, :]
    sel = np.zeros((14, 3, 30), np.float32)
    for j in range(14):
        for dx in range(3):
            sel[j, dx, 2 * j + dx] = 1.0
    a1 = jnp.einsum('yxc,jxm->jcym', w1p, jnp.asarray(sel)).reshape(448, 90)
    c1b = jnp.tile(conv1_b.reshape(1, 32), (14, 1)).reshape(448, 1)
    w2f = conv2_w.reshape(288, 64).T
    c2b = conv2_b.reshape(64, 1)
    enct = enc_w.T
    encb = enc_b.reshape(64, 1)

    dct = (codebook @ dec_w).T
    decb = dec_b.reshape(1568, 1)
    t1 = lambda dy, dx: deconv1_w[dy, dx].T
    ws1 = jnp.concatenate([
        t1(2, 2), t1(2, 1), t1(1, 2), t1(1, 1),
        t1(0, 2), t1(0, 1),
        t1(2, 0), t1(1, 0),
        t1(0, 0),
    ], axis=0)
    b1 = deconv1_b.reshape(64, 1)

    ws2_rows = []
    for ey in _E2:
        for ex in _E2:
            for (_, ay) in ey['feeds']:
                for (_, ax) in ex['feeds']:
                    ws2_rows.append(deconv2_w[ay, ax].T)
    ws2 = jnp.concatenate(ws2_rows, axis=0)
    b2 = deconv2_b.reshape(32, 1)

    w3 = jnp.zeros((16, 1152), f32)
    for ie in range(6):
        for je in range(6):
            s36 = ie * 6 + je
            for (py, ty) in _E3[ie]['feeds']:
                for (px, tx) in _E3[je]['feeds']:
                    w3 = w3.at[py * 4 + px, s36 * 32:(s36 + 1) * 32].set(
                        deconv3_w[ty, tx, :, 0])
    b3 = deconv3_b.reshape(1, 1)

    perm = np.zeros((784, 784), np.float32)
    for gy in range(7):
        for gx in range(7):
            for py in range(4):
                for px in range(4):
                    perm[(4 * gy + py) * 28 + 4 * gx + px,
                         (gy * 7 + gx) * 16 + py * 4 + px] = 1.0
    perm = jnp.asarray(perm)

    x2d = inputs.reshape(batch, 784).T

    full = lambda shape: pl.BlockSpec(shape, lambda i: tuple(0 for _ in shape))
    zt_all, idx_all = pl.pallas_call(
        _enc_body,
        grid=(nblk,),
        in_specs=[
            pl.BlockSpec((784, BB), lambda i: (0, i)),
            full((448, 90)), full((448, 1)), full((64, 288)), full((64, 1)),
            full((64, 3136)), full((64, 1)), full((NUM_EMBED, EMBED_DIM)),
        ],
        out_specs=[
            pl.BlockSpec((EMBED_DIM, BB), lambda i: (0, i)),
            pl.BlockSpec((1, 1, BB), lambda i: (i, 0, 0)),
        ],
        out_shape=[
            jax.ShapeDtypeStruct((EMBED_DIM, batch), f32),
            jax.ShapeDtypeStruct((nblk, 1, BB), jnp.int32),
        ],
        scratch_shapes=[
            pltpu.VMEM((900, BB), f32), pltpu.VMEM((7200, BB), f32),
            pltpu.VMEM((288, NL), f32), pltpu.VMEM((3136, BB), f32),
        ],
        compiler_params=pltpu.CompilerParams(
            vmem_limit_bytes=120 * 1024 * 1024),
        interpret=_INTERPRET,
    )(x2d, a1, c1b, w2f, c2b, enct, encb, codebook)

    q = _sc_gather(codebook, idx_all.reshape(batch))

    rec2d = pl.pallas_call(
        _dec_body,
        grid=(nblk,),
        in_specs=[
            pl.BlockSpec((1, 1, BB), lambda i: (i, 0, 0)),
            full((1568, 64)), full((1568, 1)), full((576, 32)), full((64, 1)),
            full((1152, 64)), full((32, 1)), full((16, 1152)), full((1, 1)),
            full((784, 784)),
        ],
        out_specs=pl.BlockSpec((784, BB), lambda i: (0, i)),
        out_shape=jax.ShapeDtypeStruct((784, batch), f32),
        scratch_shapes=[
            pltpu.VMEM((32, NL), f32), pltpu.VMEM((32, NL), f32),
            pltpu.VMEM((32, NL), f32), pltpu.VMEM((32, NL), f32),
            pltpu.VMEM((256, NL), f32), pltpu.VMEM((320, NL), f32),
            pltpu.VMEM((1152, NL), f32),
        ],
        compiler_params=pltpu.CompilerParams(
            vmem_limit_bytes=120 * 1024 * 1024),
        interpret=_INTERPRET,
    )(idx_all, dct, decb, ws1, b1, ws2, b2, w3, b3, perm)

    lb = min(512, batch)
    loss_acc = pl.pallas_call(
        _loss_body,
        grid=(batch // lb,),
        in_specs=[
            pl.BlockSpec((lb, 128), lambda i: (i, 0)),
            pl.BlockSpec((EMBED_DIM, lb), lambda i: (0, i)),
        ],
        out_specs=pl.BlockSpec((8, 128), lambda i: (0, 0)),
        out_shape=jax.ShapeDtypeStruct((8, 128), f32),
        interpret=_INTERPRET,
    )(q, zt_all)

    vq_loss = 2.0 * loss_acc[0, 0] / (batch * EMBED_DIM)
    reconstructed = rec2d.T.reshape(batch, 28, 28, 1)
    return (reconstructed, vq_loss)


# speedup vs baseline: 1.7115x; 1.7115x over previous
import functools

import jax
import jax.numpy as jnp
import numpy as np
from jax import lax
from jax.experimental import pallas as pl
from jax.experimental.pallas import tpu as pltpu
from jax.experimental.pallas import tpu_sc as plsc

EMBED_DIM = 64
NUM_EMBED = 64
BB = 128
NPIX7 = 49
NL = NPIX7 * BB

_INTERPRET = False


def _dot(a, b):
    return jax.lax.dot_general(a, b, (((1,), (0,)), ((), ())),
                               preferred_element_type=jnp.float32)


_E2 = [
    dict(r=0, s=0, feeds=[(1, 1), (2, 0), (0, 2)]),
    dict(r=1, s=0, feeds=[(2, 2), (3, 1)]),
    dict(r=1, s=-1, feeds=[(0, 0)]),
]
_E3 = [
    dict(p=0, s=0, feeds=[(1, 0), (0, 1)]),
    dict(p=1, s=0, feeds=[(2, 0), (1, 1), (0, 2)]),
    dict(p=2, s=0, feeds=[(3, 0), (2, 1), (1, 2)]),
    dict(p=3, s=0, feeds=[(3, 1), (2, 2)]),
    dict(p=0, s=1, feeds=[(3, 2)]),
    dict(p=3, s=-1, feeds=[(0, 0)]),
]


def _enc_body(x_ref, a1_ref, c1b_ref, w2f_ref, c2b_ref, enct_ref, encb_ref,
              cb_ref, z_ref, idx_ref, xp, h1p, p2, hft):
    i = pl.program_id(0)

    @pl.when(i == 0)
    def _zero():
        xp[:] = jnp.zeros((900, BB), jnp.float32)
        h1p[:] = jnp.zeros((7200, BB), jnp.float32)

    for y in range(28):
        xp[y * 30:y * 30 + 28, :] = x_ref[y, :, 0, :]

    a1 = a1_ref[:]
    c1b = c1b_ref[:]
    for i1 in range(14):
        rows = xp[2 * i1 * 30: 2 * i1 * 30 + 90, :]
        val = jnp.maximum(_dot(a1, rows) + c1b, 0.0)
        h1p[i1 * 480: i1 * 480 + 448, :] = val

    for ij in range(NPIX7):
        i2, j2 = divmod(ij, 7)
        for dy in range(3):
            src = ((2 * i2 + dy) * 15 + 2 * j2) * 32
            p2[dy * 96:(dy + 1) * 96, ij * BB:(ij + 1) * BB] = \
                h1p[src: src + 96, :]
    h2 = jnp.maximum(_dot(w2f_ref[:], p2[:]) + c2b_ref[:], 0.0)

    for ij in range(NPIX7):
        hft[ij * 64:(ij + 1) * 64, :] = h2[:, ij * BB:(ij + 1) * BB]
    zt = _dot(enct_ref[:], hft[:]) + encb_ref[:]
    z_ref[:] = zt

    cb = cb_ref[:]
    cbn = jnp.sum(cb * cb, axis=1, keepdims=True)
    dist = cbn - 2.0 * _dot(cb, zt)
    mn = jnp.min(dist, axis=0, keepdims=True)
    ki = jax.lax.broadcasted_iota(jnp.int32, (NUM_EMBED, BB), 0)
    cand = jnp.where(dist <= mn, ki, NUM_EMBED)
    idx = jnp.min(cand, axis=0, keepdims=True)
    idx_ref[:] = idx.reshape(1, 1, BB)


def _shift2d(src, su, sv):
    plan = []
    v0 = max(0, -sv)
    v1 = 7 - max(0, sv)
    for u in range(max(0, -su), 7 - max(0, su)):
        dst = (u * 7 + v0) * BB
        srcl = ((u + su) * 7 + v0 + sv) * BB
        plan.append((dst, srcl, (v1 - v0) * BB))
    return plan


def _dec_body(idx_ref, dct_ref, decb_ref, ws1_ref, b1_ref,
              ws2_ref, b2_ref, w3_ref, b3_ref, perm_ref,
              rec_ref, d1l, shm0, sh0m, shmm, p1ph, psh, sall3):
    i = pl.program_id(0)

    @pl.when(i == 0)
    def _zero():
        shm0[:] = jnp.zeros((32, NL), jnp.float32)
        sh0m[:] = jnp.zeros((32, NL), jnp.float32)
        shmm[:] = jnp.zeros((32, NL), jnp.float32)
        psh[:] = jnp.zeros((320, NL), jnp.float32)
        sall3[:] = jnp.zeros((1152, NL), jnp.float32)

    ki = jax.lax.broadcasted_iota(jnp.int32, (NUM_EMBED, BB), 0)
    oh = (ki == idx_ref[0]).astype(jnp.float32)
    d1t = jnp.maximum(_dot(dct_ref[:], oh) + decb_ref[:], 0.0)

    for p in range(NPIX7):
        d1l[:, p * BB:(p + 1) * BB] = d1t[p * 32:(p + 1) * 32, :]

    for dst, src, w in _shift2d(None, -1, 0):
        shm0[:, dst:dst + w] = d1l[:, src:src + w]
    for dst, src, w in _shift2d(None, 0, -1):
        sh0m[:, dst:dst + w] = d1l[:, src:src + w]
    for dst, src, w in _shift2d(None, -1, -1):
        shmm[:, dst:dst + w] = d1l[:, src:src + w]

    ws1 = ws1_ref[:]
    r00 = _dot(ws1[0:256, :], d1l[:])
    rm0 = _dot(ws1[256:384, :], shm0[:])
    r0m = _dot(ws1[384:512, :], sh0m[:])
    rmm = _dot(ws1[512:576, :], shmm[:])
    b1 = b1_ref[:]
    p1ph[0:64, :] = jnp.maximum(
        r00[0:64] + rm0[0:64] + r0m[0:64] + rmm + b1, 0.0)
    p1ph[64:128, :] = jnp.maximum(r00[64:128] + rm0[64:128] + b1, 0.0)
    p1ph[128:192, :] = jnp.maximum(r00[128:192] + r0m[64:128] + b1, 0.0)
    p1ph[192:256, :] = jnp.maximum(r00[192:256] + b1, 0.0)

    shifted = {(0, 1, 0, -1): 0, (1, 1, 0, -1): 1,
               (1, 0, -1, 0): 2, (1, 1, -1, 0): 3, (1, 1, -1, -1): 4}
    for (ry, rx, su, sv), slot in shifted.items():
        src_rows = (ry * 2 + rx) * 64
        for dst, src, w in _shift2d(None, su, sv):
            psh[slot * 64:(slot + 1) * 64, dst:dst + w] = \
                p1ph[src_rows:src_rows + 64, src:src + w]

    b2 = b2_ref[:]
    ws2 = ws2_ref[:]
    acc = {}
    off = 0
    for ey in _E2:
        for ex in _E2:
            ry, rx = ey['r'], ex['r']
            su, sv = ey['s'], ex['s']
            n = len(ey['feeds']) * len(ex['feeds'])
            if su == 0 and sv == 0:
                src = p1ph[(ry * 2 + rx) * 64:(ry * 2 + rx) * 64 + 64, :]
            else:
                slot = shifted[(ry, rx, su, sv)]
                src = psh[slot * 64:(slot + 1) * 64, :]
            res = _dot(ws2[off:off + 32 * n, :], src)
            off += 32 * n
            k = 0
            for (py, _) in ey['feeds']:
                for (px, _) in ex['feeds']:
                    acc.setdefault((py, px), []).append(
                        res[k * 32:(k + 1) * 32])
                    k += 1
    for (py, px), parts in acc.items():
        tot = parts[0]
        for q in parts[1:]:
            tot = tot + q
        slot = py * 6 + px
        sall3[slot * 32:(slot + 1) * 32, :] = jnp.maximum(tot + b2, 0.0)

    for ie in range(6):
        for je in range(6):
            if ie < 4 and je < 4:
                continue
            sy, sx = _E3[ie]['p'], _E3[je]['p']
            su, sv = _E3[ie]['s'], _E3[je]['s']
            srow = (sy * 6 + sx) * 32
            drow = (ie * 6 + je) * 32
            for dst, src, w in _shift2d(None, su, sv):
                sall3[drow:drow + 32, dst:dst + w] = \
                    sall3[srow:srow + 32, src:src + w]

    r3 = jax.nn.sigmoid(_dot(w3_ref[:], sall3[:]) + b3_ref[:])

    cat = [r3[:, g * BB:(g + 1) * BB] for g in range(NPIX7)]
    cat2 = jnp.concatenate(cat, axis=0)
    rec_ref[:] = _dot(perm_ref[:], cat2)


def _loss_body(q_ref, zt_ref, loss_ref):
    i = pl.program_id(0)

    @pl.when(i == 0)
    def _zero():
        loss_ref[:] = jnp.zeros((8, 128), jnp.float32)

    qt = q_ref[:, 0:EMBED_DIM].T
    diff = qt - zt_ref[:]
    loss_ref[:] += jnp.sum(diff * diff)


def _sc_gather(codebook, idx):
    table = jnp.pad(codebook, ((0, 0), (0, 128 - EMBED_DIM)))
    info = plsc.get_sparse_core_info()
    nw = info.num_cores * info.num_subcores
    n = idx.shape[0]
    bpw = n // nw
    mesh = plsc.VectorSubcoreMesh(core_axis_name="c", subcore_axis_name="s")

    @functools.partial(
        pl.kernel, mesh=mesh,
        out_type=jax.ShapeDtypeStruct((n, 128), jnp.float32),
        scratch_types=[
            pltpu.VMEM((bpw,), jnp.int32),
            pltpu.VMEM((bpw, 128), jnp.float32),
            pltpu.SemaphoreType.DMA,
        ],
    )
    def k(table_hbm, idx_hbm, out_hbm, idx_v, rows_v, sem):
        wid = lax.axis_index("s") * info.num_cores + lax.axis_index("c")
        base = wid * bpw
        pltpu.sync_copy(idx_hbm.at[pl.ds(base, bpw)], idx_v)
        pltpu.async_copy(table_hbm.at[idx_v], rows_v, sem).wait()
        pltpu.sync_copy(rows_v, out_hbm.at[pl.ds(base, bpw)])

    return k(table, idx)


def kernel(inputs, conv1_w, conv1_b, conv2_w, conv2_b, enc_w, enc_b, codebook,
           dec_w, dec_b, deconv1_w, deconv1_b, deconv2_w, deconv2_b,
           deconv3_w, deconv3_b):
    batch = inputs.shape[0]
    nblk = batch // BB
    f32 = jnp.float32

    w1p = conv1_w[:, :, 0, :]
    sel = np.zeros((14, 3, 30), np.float32)
    for j in range(14):
        for dx in range(3):
            sel[j, dx, 2 * j + dx] = 1.0
    a1 = jnp.einsum('yxc,jxm->jcym', w1p, jnp.asarray(sel)).reshape(448, 90)
    c1b = jnp.tile(conv1_b.reshape(1, 32), (14, 1)).reshape(448, 1)
    w2f = conv2_w.reshape(288, 64).T
    c2b = conv2_b.reshape(64, 1)
    enct = enc_w.T
    encb = enc_b.reshape(64, 1)

    dct = (codebook @ dec_w).T
    decb = dec_b.reshape(1568, 1)
    t1 = lambda dy, dx: deconv1_w[dy, dx].T
    ws1 = jnp.concatenate([
        t1(2, 2), t1(2, 1), t1(1, 2), t1(1, 1),
        t1(0, 2), t1(0, 1),
        t1(2, 0), t1(1, 0),
        t1(0, 0),
    ], axis=0)
    b1 = deconv1_b.reshape(64, 1)

    ws2_rows = []
    for ey in _E2:
        for ex in _E2:
            for (_, ay) in ey['feeds']:
                for (_, ax) in ex['feeds']:
                    ws2_rows.append(deconv2_w[ay, ax].T)
    ws2 = jnp.concatenate(ws2_rows, axis=0)
    b2 = deconv2_b.reshape(32, 1)

    sel3 = np.zeros((16, 36, 3, 3), np.float32)
    for ie in range(6):
        for je in range(6):
            s36 = ie * 6 + je
            for (py, ty) in _E3[ie]['feeds']:
                for (px, tx) in _E3[je]['feeds']:
                    sel3[py * 4 + px, s36, ty, tx] = 1.0
    w3 = jnp.einsum('pstu,tuc->psc', jnp.asarray(sel3),
                    deconv3_w[:, :, :, 0]).reshape(16, 1152)
    b3 = deconv3_b.reshape(1, 1)

    perm = np.zeros((784, 784), np.float32)
    for gy in range(7):
        for gx in range(7):
            for py in range(4):
                for px in range(4):
                    perm[(4 * gy + py) * 28 + 4 * gx + px,
                         (gy * 7 + gx) * 16 + py * 4 + px] = 1.0
    perm = jnp.asarray(perm)

    x2d = inputs.transpose(1, 2, 3, 0)

    full = lambda shape: pl.BlockSpec(shape, lambda i: tuple(0 for _ in shape))
    zt_all, idx_all = pl.pallas_call(
        _enc_body,
        grid=(nblk,),
        in_specs=[
            pl.BlockSpec((28, 28, 1, BB), lambda i: (0, 0, 0, i)),
            full((448, 90)), full((448, 1)), full((64, 288)), full((64, 1)),
            full((64, 3136)), full((64, 1)), full((NUM_EMBED, EMBED_DIM)),
        ],
        out_specs=[
            pl.BlockSpec((EMBED_DIM, BB), lambda i: (0, i)),
            pl.BlockSpec((1, 1, BB), lambda i: (i, 0, 0)),
        ],
        out_shape=[
            jax.ShapeDtypeStruct((EMBED_DIM, batch), f32),
            jax.ShapeDtypeStruct((nblk, 1, BB), jnp.int32),
        ],
        scratch_shapes=[
            pltpu.VMEM((900, BB), f32), pltpu.VMEM((7200, BB), f32),
            pltpu.VMEM((288, NL), f32), pltpu.VMEM((3136, BB), f32),
        ],
        compiler_params=pltpu.CompilerParams(
            vmem_limit_bytes=120 * 1024 * 1024),
        interpret=_INTERPRET,
    )(x2d, a1, c1b, w2f, c2b, enct, encb, codebook)

    q = _sc_gather(codebook, idx_all.reshape(batch))

    rec2d = pl.pallas_call(
        _dec_body,
        grid=(nblk,),
        in_specs=[
            pl.BlockSpec((1, 1, BB), lambda i: (i, 0, 0)),
            full((1568, 64)), full((1568, 1)), full((576, 32)), full((64, 1)),
            full((1152, 64)), full((32, 1)), full((16, 1152)), full((1, 1)),
            full((784, 784)),
        ],
        out_specs=pl.BlockSpec((784, BB), lambda i: (0, i)),
        out_shape=jax.ShapeDtypeStruct((784, batch), f32),
        scratch_shapes=[
            pltpu.VMEM((32, NL), f32), pltpu.VMEM((32, NL), f32),
            pltpu.VMEM((32, NL), f32), pltpu.VMEM((32, NL), f32),
            pltpu.VMEM((256, NL), f32), pltpu.VMEM((320, NL), f32),
            pltpu.VMEM((1152, NL), f32),
        ],
        compiler_params=pltpu.CompilerParams(
            vmem_limit_bytes=120 * 1024 * 1024),
        interpret=_INTERPRET,
    )(idx_all, dct, decb, ws1, b1, ws2, b2, w3, b3, perm)

    lb = min(512, batch)
    loss_acc = pl.pallas_call(
        _loss_body,
        grid=(batch // lb,),
        in_specs=[
            pl.BlockSpec((lb, 128), lambda i: (i, 0)),
            pl.BlockSpec((EMBED_DIM, lb), lambda i: (0, i)),
        ],
        out_specs=pl.BlockSpec((8, 128), lambda i: (0, 0)),
        out_shape=jax.ShapeDtypeStruct((8, 128), f32),
        interpret=_INTERPRET,
    )(q, zt_all)

    vq_loss = 2.0 * loss_acc[0, 0] / (batch * EMBED_DIM)
    reconstructed = rec2d.T.reshape(batch, 28, 28, 1)
    return (reconstructed, vq_loss)


# speedup vs baseline: 1.8654x; 1.0899x over previous
import functools

import jax
import jax.numpy as jnp
import numpy as np
from jax import lax
from jax.experimental import pallas as pl
from jax.experimental.pallas import tpu as pltpu
from jax.experimental.pallas import tpu_sc as plsc

EMBED_DIM = 64
NUM_EMBED = 64
BB = 128
NPIX7 = 49
NL = NPIX7 * BB

_INTERPRET = False


def _dot(a, b):
    return jax.lax.dot_general(a, b, (((1,), (0,)), ((), ())),
                               preferred_element_type=jnp.float32)


_E2 = [
    dict(r=0, s=0, feeds=[(1, 1), (2, 0), (0, 2)]),
    dict(r=1, s=0, feeds=[(2, 2), (3, 1)]),
    dict(r=1, s=-1, feeds=[(0, 0)]),
]
_E3 = [
    dict(p=0, s=0, feeds=[(1, 0), (0, 1)]),
    dict(p=1, s=0, feeds=[(2, 0), (1, 1), (0, 2)]),
    dict(p=2, s=0, feeds=[(3, 0), (2, 1), (1, 2)]),
    dict(p=3, s=0, feeds=[(3, 1), (2, 2)]),
    dict(p=0, s=1, feeds=[(3, 2)]),
    dict(p=3, s=-1, feeds=[(0, 0)]),
]


def _enc_body(x_ref, a1_ref, c1b_ref, w2f_ref, c2b_ref, enct_ref, encb_ref,
              cb_ref, z_ref, idx_ref, xp, h1p, p2, hft):
    i = pl.program_id(0)

    @pl.when(i == 0)
    def _zero():
        xp[:] = jnp.zeros((900, BB), jnp.float32)
        h1p[:] = jnp.zeros((7200, BB), jnp.float32)

    for y in range(28):
        xp[y * 30:y * 30 + 28, :] = x_ref[y, :, 0, :]

    a1 = a1_ref[:]
    c1b = c1b_ref[:]
    for i1 in range(14):
        rows = xp[2 * i1 * 30: 2 * i1 * 30 + 90, :]
        val = jnp.maximum(_dot(a1, rows) + c1b, 0.0)
        h1p[i1 * 480: i1 * 480 + 448, :] = val

    for ij in range(NPIX7):
        i2, j2 = divmod(ij, 7)
        for dy in range(3):
            src = ((2 * i2 + dy) * 15 + 2 * j2) * 32
            p2[dy * 96:(dy + 1) * 96, ij * BB:(ij + 1) * BB] = \
                h1p[src: src + 96, :]
    h2 = jnp.maximum(_dot(w2f_ref[:], p2[:]) + c2b_ref[:], 0.0)

    for ij in range(NPIX7):
        hft[ij * 64:(ij + 1) * 64, :] = h2[:, ij * BB:(ij + 1) * BB]
    zt = _dot(enct_ref[:], hft[:]) + encb_ref[:]
    z_ref[:] = zt

    cb = cb_ref[:]
    cbn = jnp.sum(cb * cb, axis=1, keepdims=True)
    dist = cbn - 2.0 * _dot(cb, zt)
    mn = jnp.min(dist, axis=0, keepdims=True)
    ki = jax.lax.broadcasted_iota(jnp.int32, (NUM_EMBED, BB), 0)
    cand = jnp.where(dist <= mn, ki, NUM_EMBED)
    idx = jnp.min(cand, axis=0, keepdims=True)
    idx_ref[:] = idx.reshape(1, 1, BB)


def _shift2d(src, su, sv):
    plan = []
    v0 = max(0, -sv)
    v1 = 7 - max(0, sv)
    for u in range(max(0, -su), 7 - max(0, su)):
        dst = (u * 7 + v0) * BB
        srcl = ((u + su) * 7 + v0 + sv) * BB
        plan.append((dst, srcl, (v1 - v0) * BB))
    return plan


def _dec_body(idx_ref, dct_ref, decb_ref, ws1_ref, b1_ref,
              ws2_ref, b2_ref, w3_ref, b3_ref, perm_ref,
              rec_ref, d1l, shm0, sh0m, shmm, p1ph, psh, sall3):
    i = pl.program_id(0)

    @pl.when(i == 0)
    def _zero():
        shm0[:] = jnp.zeros((32, NL), jnp.float32)
        sh0m[:] = jnp.zeros((32, NL), jnp.float32)
        shmm[:] = jnp.zeros((32, NL), jnp.float32)
        psh[:] = jnp.zeros((320, NL), jnp.float32)
        sall3[:] = jnp.zeros((1152, NL), jnp.float32)

    ki = jax.lax.broadcasted_iota(jnp.int32, (NUM_EMBED, BB), 0)
    oh = (ki == idx_ref[0]).astype(jnp.float32)
    d1t = jnp.maximum(_dot(dct_ref[:], oh) + decb_ref[:], 0.0)

    for p in range(NPIX7):
        d1l[:, p * BB:(p + 1) * BB] = d1t[p * 32:(p + 1) * 32, :]

    for dst, src, w in _shift2d(None, -1, 0):
        shm0[:, dst:dst + w] = d1l[:, src:src + w]
    for dst, src, w in _shift2d(None, 0, -1):
        sh0m[:, dst:dst + w] = d1l[:, src:src + w]
    for dst, src, w in _shift2d(None, -1, -1):
        shmm[:, dst:dst + w] = d1l[:, src:src + w]

    ws1 = ws1_ref[:]
    r00 = _dot(ws1[0:256, :], d1l[:])
    rm0 = _dot(ws1[256:384, :], shm0[:])
    r0m = _dot(ws1[384:512, :], sh0m[:])
    rmm = _dot(ws1[512:576, :], shmm[:])
    b1 = b1_ref[:]
    p1ph[0:64, :] = jnp.maximum(
        r00[0:64] + rm0[0:64] + r0m[0:64] + rmm + b1, 0.0)
    p1ph[64:128, :] = jnp.maximum(r00[64:128] + rm0[64:128] + b1, 0.0)
    p1ph[128:192, :] = jnp.maximum(r00[128:192] + r0m[64:128] + b1, 0.0)
    p1ph[192:256, :] = jnp.maximum(r00[192:256] + b1, 0.0)

    shifted = {(0, 1, 0, -1): 0, (1, 1, 0, -1): 1,
               (1, 0, -1, 0): 2, (1, 1, -1, 0): 3, (1, 1, -1, -1): 4}
    for (ry, rx, su, sv), slot in shifted.items():
        src_rows = (ry * 2 + rx) * 64
        for dst, src, w in _shift2d(None, su, sv):
            psh[slot * 64:(slot + 1) * 64, dst:dst + w] = \
                p1ph[src_rows:src_rows + 64, src:src + w]

    b2 = b2_ref[:]
    ws2 = ws2_ref[:]
    acc = {}
    off = 0
    for ey in _E2:
        for ex in _E2:
            ry, rx = ey['r'], ex['r']
            su, sv = ey['s'], ex['s']
            n = len(ey['feeds']) * len(ex['feeds'])
            if su == 0 and sv == 0:
                src = p1ph[(ry * 2 + rx) * 64:(ry * 2 + rx) * 64 + 64, :]
            else:
                slot = shifted[(ry, rx, su, sv)]
                src = psh[slot * 64:(slot + 1) * 64, :]
            res = _dot(ws2[off:off + 32 * n, :], src)
            off += 32 * n
            k = 0
            for (py, _) in ey['feeds']:
                for (px, _) in ex['feeds']:
                    acc.setdefault((py, px), []).append(
                        res[k * 32:(k + 1) * 32])
                    k += 1
    for (py, px), parts in acc.items():
        tot = parts[0]
        for q in parts[1:]:
            tot = tot + q
        slot = py * 6 + px
        sall3[slot * 32:(slot + 1) * 32, :] = jnp.maximum(tot + b2, 0.0)

    for ie in range(6):
        for je in range(6):
            if ie < 4 and je < 4:
                continue
            sy, sx = _E3[ie]['p'], _E3[je]['p']
            su, sv = _E3[ie]['s'], _E3[je]['s']
            srow = (sy * 6 + sx) * 32
            drow = (ie * 6 + je) * 32
            for dst, src, w in _shift2d(None, su, sv):
                sall3[drow:drow + 32, dst:dst + w] = \
                    sall3[srow:srow + 32, src:src + w]

    r3 = jax.nn.sigmoid(_dot(w3_ref[:], sall3[:]) + b3_ref[:])

    cat = [r3[:, g * BB:(g + 1) * BB] for g in range(NPIX7)]
    cat2 = jnp.concatenate(cat, axis=0)
    rec_ref[:] = _dot(perm_ref[:], cat2)


def _loss_body(q_ref, zt_ref, loss_ref):
    i = pl.program_id(0)

    @pl.when(i == 0)
    def _zero():
        loss_ref[:] = jnp.zeros((8, 128), jnp.float32)

    qt = q_ref[:, 0:EMBED_DIM].T
    diff = qt - zt_ref[:]
    loss_ref[:] += jnp.sum(diff * diff)


def _sc_gather(codebook, idx):
    table = jnp.pad(codebook, ((0, 0), (0, 128 - EMBED_DIM)))
    info = plsc.get_sparse_core_info()
    nw = info.num_cores * info.num_subcores
    n = idx.shape[0]
    bpw = n // nw
    mesh = plsc.VectorSubcoreMesh(core_axis_name="c", subcore_axis_name="s")

    @functools.partial(
        pl.kernel, mesh=mesh,
        out_type=jax.ShapeDtypeStruct((n, 128), jnp.float32),
        scratch_types=[
            pltpu.VMEM((bpw,), jnp.int32),
            pltpu.VMEM((bpw, 128), jnp.float32),
            pltpu.SemaphoreType.DMA,
        ],
    )
    def k(table_hbm, idx_hbm, out_hbm, idx_v, rows_v, sem):
        wid = lax.axis_index("s") * info.num_cores + lax.axis_index("c")
        base = wid * bpw
        pltpu.sync_copy(idx_hbm.at[pl.ds(base, bpw)], idx_v)
        pltpu.async_copy(table_hbm.at[idx_v], rows_v, sem).wait()
        pltpu.sync_copy(rows_v, out_hbm.at[pl.ds(base, bpw)])

    return k(table, idx)


def kernel(inputs, conv1_w, conv1_b, conv2_w, conv2_b, enc_w, enc_b, codebook,
           dec_w, dec_b, deconv1_w, deconv1_b, deconv2_w, deconv2_b,
           deconv3_w, deconv3_b):
    batch = inputs.shape[0]
    nblk = batch // BB
    f32 = jnp.float32

    w1p = conv1_w[:, :, 0, :]
    sel = np.zeros((14, 3, 30), np.float32)
    for j in range(14):
        for dx in range(3):
            sel[j, dx, 2 * j + dx] = 1.0
    a1 = jnp.einsum('yxc,jxm->jcym', w1p, jnp.asarray(sel)).reshape(448, 90)
    c1b = jnp.tile(conv1_b.reshape(1, 32), (14, 1)).reshape(448, 1)
    w2f = conv2_w.reshape(288, 64).T
    c2b = conv2_b.reshape(64, 1)
    enct = enc_w.T
    encb = enc_b.reshape(64, 1)

    dct = (codebook @ dec_w).T
    decb = dec_b.reshape(1568, 1)
    taps1 = [(2, 2), (2, 1), (1, 2), (1, 1),
             (0, 2), (0, 1),
             (2, 0), (1, 0),
             (0, 0)]
    sel1 = np.zeros((9, 3, 3), np.float32)
    for r, (dy, dx) in enumerate(taps1):
        sel1[r, dy, dx] = 1.0
    ws1 = jnp.einsum('ryx,yxio->roi', jnp.asarray(sel1),
                     deconv1_w).reshape(576, 32)
    b1 = deconv1_b.reshape(64, 1)

    taps2 = []
    for ey in _E2:
        for ex in _E2:
            for (_, ay) in ey['feeds']:
                for (_, ax) in ex['feeds']:
                    taps2.append((ay, ax))
    sel2 = np.zeros((36, 3, 3), np.float32)
    for r, (ay, ax) in enumerate(taps2):
        sel2[r, ay, ax] = 1.0
    ws2 = jnp.einsum('ryx,yxio->roi', jnp.asarray(sel2),
                     deconv2_w).reshape(1152, 64)
    b2 = deconv2_b.reshape(32, 1)

    sel3 = np.zeros((16, 36, 3, 3), np.float32)
    for ie in range(6):
        for je in range(6):
            s36 = ie * 6 + je
            for (py, ty) in _E3[ie]['feeds']:
                for (px, tx) in _E3[je]['feeds']:
                    sel3[py * 4 + px, s36, ty, tx] = 1.0
    w3 = jnp.einsum('pstu,tuc->psc', jnp.asarray(sel3),
                    deconv3_w[:, :, :, 0]).reshape(16, 1152)
    b3 = deconv3_b.reshape(1, 1)

    perm = np.zeros((784, 784), np.float32)
    for gy in range(7):
        for gx in range(7):
            for py in range(4):
                for px in range(4):
                    perm[(4 * gy + py) * 28 + 4 * gx + px,
                         (gy * 7 + gx) * 16 + py * 4 + px] = 1.0
    perm = jnp.asarray(perm)

    x2d = inputs.transpose(1, 2, 3, 0)

    full = lambda shape: pl.BlockSpec(shape, lambda i: tuple(0 for _ in shape))
    zt_all, idx_all = pl.pallas_call(
        _enc_body,
        grid=(nblk,),
        in_specs=[
            pl.BlockSpec((28, 28, 1, BB), lambda i: (0, 0, 0, i)),
            full((448, 90)), full((448, 1)), full((64, 288)), full((64, 1)),
            full((64, 3136)), full((64, 1)), full((NUM_EMBED, EMBED_DIM)),
        ],
        out_specs=[
            pl.BlockSpec((EMBED_DIM, BB), lambda i: (0, i)),
            pl.BlockSpec((1, 1, BB), lambda i: (i, 0, 0)),
        ],
        out_shape=[
            jax.ShapeDtypeStruct((EMBED_DIM, batch), f32),
            jax.ShapeDtypeStruct((nblk, 1, BB), jnp.int32),
        ],
        scratch_shapes=[
            pltpu.VMEM((900, BB), f32), pltpu.VMEM((7200, BB), f32),
            pltpu.VMEM((288, NL), f32), pltpu.VMEM((3136, BB), f32),
        ],
        compiler_params=pltpu.CompilerParams(
            vmem_limit_bytes=120 * 1024 * 1024),
        interpret=_INTERPRET,
    )(x2d, a1, c1b, w2f, c2b, enct, encb, codebook)

    q = _sc_gather(codebook, idx_all.reshape(batch))

    rec2d = pl.pallas_call(
        _dec_body,
        grid=(nblk,),
        in_specs=[
            pl.BlockSpec((1, 1, BB), lambda i: (i, 0, 0)),
            full((1568, 64)), full((1568, 1)), full((576, 32)), full((64, 1)),
            full((1152, 64)), full((32, 1)), full((16, 1152)), full((1, 1)),
            full((784, 784)),
        ],
        out_specs=pl.BlockSpec((784, BB), lambda i: (0, i)),
        out_shape=jax.ShapeDtypeStruct((784, batch), f32),
        scratch_shapes=[
            pltpu.VMEM((32, NL), f32), pltpu.VMEM((32, NL), f32),
            pltpu.VMEM((32, NL), f32), pltpu.VMEM((32, NL), f32),
            pltpu.VMEM((256, NL), f32), pltpu.VMEM((320, NL), f32),
            pltpu.VMEM((1152, NL), f32),
        ],
        compiler_params=pltpu.CompilerParams(
            vmem_limit_bytes=120 * 1024 * 1024),
        interpret=_INTERPRET,
    )(idx_all, dct, decb, ws1, b1, ws2, b2, w3, b3, perm)

    lb = min(512, batch)
    loss_acc = pl.pallas_call(
        _loss_body,
        grid=(batch // lb,),
        in_specs=[
            pl.BlockSpec((lb, 128), lambda i: (i, 0)),
            pl.BlockSpec((EMBED_DIM, lb), lambda i: (0, i)),
        ],
        out_specs=pl.BlockSpec((8, 128), lambda i: (0, 0)),
        out_shape=jax.ShapeDtypeStruct((8, 128), f32),
        interpret=_INTERPRET,
    )(q, zt_all)

    vq_loss = 2.0 * loss_acc[0, 0] / (batch * EMBED_DIM)
    reconstructed = jax.lax.reshape(rec2d, (batch, 28, 28, 1),
                                    dimensions=(1, 0))
    return (reconstructed, vq_loss)


# speedup vs baseline: 1.9492x; 1.0449x over previous
import functools

import jax
import jax.numpy as jnp
import numpy as np
from jax import lax
from jax.experimental import pallas as pl
from jax.experimental.pallas import tpu as pltpu
from jax.experimental.pallas import tpu_sc as plsc

EMBED_DIM = 64
NUM_EMBED = 64
BB = 128
NPIX7 = 49
NL = NPIX7 * BB

_INTERPRET = False


def _dot(a, b):
    return jax.lax.dot_general(a, b, (((1,), (0,)), ((), ())),
                               preferred_element_type=jnp.float32)


_E2 = [
    dict(r=0, s=0, feeds=[(1, 1), (2, 0), (0, 2)]),
    dict(r=1, s=0, feeds=[(2, 2), (3, 1)]),
    dict(r=1, s=-1, feeds=[(0, 0)]),
]
_E3 = [
    dict(p=0, s=0, feeds=[(1, 0), (0, 1)]),
    dict(p=1, s=0, feeds=[(2, 0), (1, 1), (0, 2)]),
    dict(p=2, s=0, feeds=[(3, 0), (2, 1), (1, 2)]),
    dict(p=3, s=0, feeds=[(3, 1), (2, 2)]),
    dict(p=0, s=1, feeds=[(3, 2)]),
    dict(p=3, s=-1, feeds=[(0, 0)]),
]


def _enc_body(x_ref, a1_ref, c1b_ref, w2f_ref, c2b_ref, enct_ref, encb_ref,
              cb_ref, z_ref, idx_ref, xp, h1p, p2, hft):
    i = pl.program_id(0)

    @pl.when(i == 0)
    def _zero():
        xp[:] = jnp.zeros((900, BB), jnp.float32)
        h1p[:] = jnp.zeros((7200, BB), jnp.float32)

    for y in range(28):
        xp[y * 30:y * 30 + 28, :] = x_ref[y, :, 0, :]

    a1 = a1_ref[:]
    c1b = c1b_ref[:]
    for i1 in range(14):
        rows = xp[2 * i1 * 30: 2 * i1 * 30 + 90, :]
        val = jnp.maximum(_dot(a1, rows) + c1b, 0.0)
        h1p[i1 * 480: i1 * 480 + 448, :] = val

    for ij in range(NPIX7):
        i2, j2 = divmod(ij, 7)
        for dy in range(3):
            src = ((2 * i2 + dy) * 15 + 2 * j2) * 32
            p2[dy * 96:(dy + 1) * 96, ij * BB:(ij + 1) * BB] = \
                h1p[src: src + 96, :]
    h2 = jnp.maximum(_dot(w2f_ref[:], p2[:]) + c2b_ref[:], 0.0)

    for ij in range(NPIX7):
        hft[ij * 64:(ij + 1) * 64, :] = h2[:, ij * BB:(ij + 1) * BB]
    zt = _dot(enct_ref[:], hft[:]) + encb_ref[:]
    z_ref[:] = zt

    cb = cb_ref[:]
    cbn = jnp.sum(cb * cb, axis=1, keepdims=True)
    dist = cbn - 2.0 * _dot(cb, zt)
    mn = jnp.min(dist, axis=0, keepdims=True)
    ki = jax.lax.broadcasted_iota(jnp.int32, (NUM_EMBED, BB), 0)
    cand = jnp.where(dist <= mn, ki, NUM_EMBED)
    idx = jnp.min(cand, axis=0, keepdims=True)
    idx_ref[:] = idx.reshape(1, 1, BB)


def _shift2d(src, su, sv):
    plan = []
    v0 = max(0, -sv)
    v1 = 7 - max(0, sv)
    for u in range(max(0, -su), 7 - max(0, su)):
        dst = (u * 7 + v0) * BB
        srcl = ((u + su) * 7 + v0 + sv) * BB
        plan.append((dst, srcl, (v1 - v0) * BB))
    return plan


def _dec_body(idx_ref, dct_ref, decb_ref, ws1_ref, b1_ref,
              ws2_ref, b2_ref, w3_ref, b3_ref, perm_ref,
              rec_ref, srcs1, srcs2, sall3):
    i = pl.program_id(0)

    @pl.when(i == 0)
    def _zero():
        srcs1[:] = jnp.zeros((128, NL), jnp.float32)
        srcs2[:] = jnp.zeros((576, NL), jnp.float32)
        sall3[:] = jnp.zeros((1152, NL), jnp.float32)

    ki = jax.lax.broadcasted_iota(jnp.int32, (NUM_EMBED, BB), 0)
    oh = (ki == idx_ref[0]).astype(jnp.float32)
    d1t = jnp.maximum(_dot(dct_ref[:], oh) + decb_ref[:], 0.0)

    for p in range(NPIX7):
        srcs1[0:32, p * BB:(p + 1) * BB] = d1t[p * 32:(p + 1) * 32, :]
    for r, (su, sv) in enumerate([(-1, 0), (0, -1), (-1, -1)]):
        for dst, src, w in _shift2d(None, su, sv):
            srcs1[(r + 1) * 32:(r + 2) * 32, dst:dst + w] = \
                srcs1[0:32, src:src + w]

    p1 = jnp.maximum(_dot(ws1_ref[:], srcs1[:]) + b1_ref[:], 0.0)
    srcs2[0:256, :] = p1

    shifted = {(0, 1, 0, -1): 0, (1, 1, 0, -1): 1,
               (1, 0, -1, 0): 2, (1, 1, -1, 0): 3, (1, 1, -1, -1): 4}
    for (ry, rx, su, sv), slot in shifted.items():
        src_rows = (ry * 2 + rx) * 64
        for dst, src, w in _shift2d(None, su, sv):
            srcs2[256 + slot * 64:256 + (slot + 1) * 64, dst:dst + w] = \
                srcs2[src_rows:src_rows + 64, src:src + w]

    r2 = jnp.maximum(_dot(ws2_ref[:], srcs2[:]) + b2_ref[:], 0.0)
    for py in range(4):
        for px in range(4):
            sall3[(py * 6 + px) * 32:(py * 6 + px) * 32 + 32, :] = \
                r2[(py * 4 + px) * 32:(py * 4 + px) * 32 + 32, :]

    for ie in range(6):
        for je in range(6):
            if ie < 4 and je < 4:
                continue
            sy, sx = _E3[ie]['p'], _E3[je]['p']
            su, sv = _E3[ie]['s'], _E3[je]['s']
            srow = (sy * 6 + sx) * 32
            drow = (ie * 6 + je) * 32
            for dst, src, w in _shift2d(None, su, sv):
                sall3[drow:drow + 32, dst:dst + w] = \
                    sall3[srow:srow + 32, src:src + w]

    r3 = jax.nn.sigmoid(_dot(w3_ref[:], sall3[:]) + b3_ref[:])

    cat = [r3[:, g * BB:(g + 1) * BB] for g in range(NPIX7)]
    cat2 = jnp.concatenate(cat, axis=0)
    rec_ref[:] = _dot(perm_ref[:], cat2)


def _loss_body(q_ref, zt_ref, loss_ref):
    i = pl.program_id(0)

    @pl.when(i == 0)
    def _zero():
        loss_ref[:] = jnp.zeros((8, 128), jnp.float32)

    qt = q_ref[:, 0:EMBED_DIM].T
    diff = qt - zt_ref[:]
    loss_ref[:] += jnp.sum(diff * diff)


def _sc_gather(codebook, idx):
    table = jnp.pad(codebook, ((0, 0), (0, 128 - EMBED_DIM)))
    info = plsc.get_sparse_core_info()
    nw = info.num_cores * info.num_subcores
    n = idx.shape[0]
    bpw = n // nw
    mesh = plsc.VectorSubcoreMesh(core_axis_name="c", subcore_axis_name="s")

    @functools.partial(
        pl.kernel, mesh=mesh,
        out_type=jax.ShapeDtypeStruct((n, 128), jnp.float32),
        scratch_types=[
            pltpu.VMEM((bpw,), jnp.int32),
            pltpu.VMEM((bpw, 128), jnp.float32),
            pltpu.SemaphoreType.DMA,
        ],
    )
    def k(table_hbm, idx_hbm, out_hbm, idx_v, rows_v, sem):
        wid = lax.axis_index("s") * info.num_cores + lax.axis_index("c")
        base = wid * bpw
        pltpu.sync_copy(idx_hbm.at[pl.ds(base, bpw)], idx_v)
        pltpu.async_copy(table_hbm.at[idx_v], rows_v, sem).wait()
        pltpu.sync_copy(rows_v, out_hbm.at[pl.ds(base, bpw)])

    return k(table, idx)


def kernel(inputs, conv1_w, conv1_b, conv2_w, conv2_b, enc_w, enc_b, codebook,
           dec_w, dec_b, deconv1_w, deconv1_b, deconv2_w, deconv2_b,
           deconv3_w, deconv3_b):
    batch = inputs.shape[0]
    nblk = batch // BB
    f32 = jnp.float32

    w1p = conv1_w[:, :, 0, :]
    sel = np.zeros((14, 3, 30), np.float32)
    for j in range(14):
        for dx in range(3):
            sel[j, dx, 2 * j + dx] = 1.0
    a1 = jnp.einsum('yxc,jxm->jcym', w1p, jnp.asarray(sel)).reshape(448, 90)
    c1b = jnp.tile(conv1_b.reshape(1, 32), (14, 1)).reshape(448, 1)
    w2f = conv2_w.reshape(288, 64).T
    c2b = conv2_b.reshape(64, 1)
    enct = enc_w.T
    encb = enc_b.reshape(64, 1)

    dct = (codebook @ dec_w).T
    decb = dec_b.reshape(1568, 1)
    feeds1 = [[(0, (2, 2)), (1, (0, 2)), (2, (2, 0)), (3, (0, 0))],
              [(0, (2, 1)), (1, (0, 1))],
              [(0, (1, 2)), (2, (1, 0))],
              [(0, (1, 1))]]
    sel1 = np.zeros((4, 4, 3, 3), np.float32)
    for p, feeds in enumerate(feeds1):
        for (r, (dy, dx)) in feeds:
            sel1[p, r, dy, dx] = 1.0
    ws1 = jnp.einsum('pryx,yxio->pori', jnp.asarray(sel1),
                     deconv1_w).reshape(256, 128)
    b1 = jnp.tile(deconv1_b.reshape(1, 64), (4, 1)).reshape(256, 1)

    src_slot2 = {(0, 0, 0, 0): 0, (0, 1, 0, 0): 1,
                 (1, 0, 0, 0): 2, (1, 1, 0, 0): 3,
                 (0, 1, 0, -1): 4, (1, 1, 0, -1): 5,
                 (1, 0, -1, 0): 6, (1, 1, -1, 0): 7, (1, 1, -1, -1): 8}
    sel2 = np.zeros((16, 9, 3, 3), np.float32)
    for ey in _E2:
        for ex in _E2:
            s = src_slot2[(ey['r'], ex['r'], ey['s'], ex['s'])]
            for (py, ty) in ey['feeds']:
                for (px, tx) in ex['feeds']:
                    sel2[py * 4 + px, s, ty, tx] = 1.0
    ws2 = jnp.einsum('psyx,yxio->posi', jnp.asarray(sel2),
                     deconv2_w).reshape(512, 576)
    b2 = jnp.tile(deconv2_b.reshape(1, 32), (16, 1)).reshape(512, 1)

    sel3 = np.zeros((16, 36, 3, 3), np.float32)
    for ie in range(6):
        for je in range(6):
            s36 = ie * 6 + je
            for (py, ty) in _E3[ie]['feeds']:
                for (px, tx) in _E3[je]['feeds']:
                    sel3[py * 4 + px, s36, ty, tx] = 1.0
    w3 = jnp.einsum('pstu,tuc->psc', jnp.asarray(sel3),
                    deconv3_w[:, :, :, 0]).reshape(16, 1152)
    b3 = deconv3_b.reshape(1, 1)

    perm = np.zeros((784, 784), np.float32)
    for gy in range(7):
        for gx in range(7):
            for py in range(4):
                for px in range(4):
                    perm[(4 * gy + py) * 28 + 4 * gx + px,
                         (gy * 7 + gx) * 16 + py * 4 + px] = 1.0
    perm = jnp.asarray(perm)

    x2d = inputs.transpose(1, 2, 3, 0)

    full = lambda shape: pl.BlockSpec(shape, lambda i: tuple(0 for _ in shape))
    zt_all, idx_all = pl.pallas_call(
        _enc_body,
        grid=(nblk,),
        in_specs=[
            pl.BlockSpec((28, 28, 1, BB), lambda i: (0, 0, 0, i)),
            full((448, 90)), full((448, 1)), full((64, 288)), full((64, 1)),
            full((64, 3136)), full((64, 1)), full((NUM_EMBED, EMBED_DIM)),
        ],
        out_specs=[
            pl.BlockSpec((EMBED_DIM, BB), lambda i: (0, i)),
            pl.BlockSpec((1, 1, BB), lambda i: (i, 0, 0)),
        ],
        out_shape=[
            jax.ShapeDtypeStruct((EMBED_DIM, batch), f32),
            jax.ShapeDtypeStruct((nblk, 1, BB), jnp.int32),
        ],
        scratch_shapes=[
            pltpu.VMEM((900, BB), f32), pltpu.VMEM((7200, BB), f32),
            pltpu.VMEM((288, NL), f32), pltpu.VMEM((3136, BB), f32),
        ],
        compiler_params=pltpu.CompilerParams(
            vmem_limit_bytes=120 * 1024 * 1024),
        interpret=_INTERPRET,
    )(x2d, a1, c1b, w2f, c2b, enct, encb, codebook)

    q = _sc_gather(codebook, idx_all.reshape(batch))

    rec2d = pl.pallas_call(
        _dec_body,
        grid=(nblk,),
        in_specs=[
            pl.BlockSpec((1, 1, BB), lambda i: (i, 0, 0)),
            full((1568, 64)), full((1568, 1)), full((256, 128)), full((256, 1)),
            full((512, 576)), full((512, 1)), full((16, 1152)), full((1, 1)),
            full((784, 784)),
        ],
        out_specs=pl.BlockSpec((784, BB), lambda i: (0, i)),
        out_shape=jax.ShapeDtypeStruct((784, batch), f32),
        scratch_shapes=[
            pltpu.VMEM((128, NL), f32), pltpu.VMEM((576, NL), f32),
            pltpu.VMEM((1152, NL), f32),
        ],
        compiler_params=pltpu.CompilerParams(
            vmem_limit_bytes=120 * 1024 * 1024),
        interpret=_INTERPRET,
    )(idx_all, dct, decb, ws1, b1, ws2, b2, w3, b3, perm)

    lb = min(512, batch)
    loss_acc = pl.pallas_call(
        _loss_body,
        grid=(batch // lb,),
        in_specs=[
            pl.BlockSpec((lb, 128), lambda i: (i, 0)),
            pl.BlockSpec((EMBED_DIM, lb), lambda i: (0, i)),
        ],
        out_specs=pl.BlockSpec((8, 128), lambda i: (0, 0)),
        out_shape=jax.ShapeDtypeStruct((8, 128), f32),
        interpret=_INTERPRET,
    )(q, zt_all)

    vq_loss = 2.0 * loss_acc[0, 0] / (batch * EMBED_DIM)
    reconstructed = jax.lax.reshape(rec2d, (batch, 28, 28, 1),
                                    dimensions=(1, 0))
    return (reconstructed, vq_loss)


# speedup vs baseline: 2.4389x; 1.2512x over previous
import functools

import jax
import jax.numpy as jnp
import numpy as np
from jax import lax
from jax.experimental import pallas as pl
from jax.experimental.pallas import tpu as pltpu
from jax.experimental.pallas import tpu_sc as plsc

EMBED_DIM = 64
NUM_EMBED = 64
BB = 128
NPIX7 = 49
NL = NPIX7 * BB

_INTERPRET = False


def _dot(a, b):
    return jax.lax.dot_general(a, b, (((1,), (0,)), ((), ())),
                               preferred_element_type=jnp.float32)


_E2 = [
    dict(r=0, s=0, feeds=[(1, 1), (2, 0), (0, 2)]),
    dict(r=1, s=0, feeds=[(2, 2), (3, 1)]),
    dict(r=1, s=-1, feeds=[(0, 0)]),
]
_E3 = [
    dict(p=0, s=0, feeds=[(1, 0), (0, 1)]),
    dict(p=1, s=0, feeds=[(2, 0), (1, 1), (0, 2)]),
    dict(p=2, s=0, feeds=[(3, 0), (2, 1), (1, 2)]),
    dict(p=3, s=0, feeds=[(3, 1), (2, 2)]),
    dict(p=0, s=1, feeds=[(3, 2)]),
    dict(p=3, s=-1, feeds=[(0, 0)]),
]


def _enc_body(x_ref, a1_ref, c1b_ref, w2f_ref, c2b_ref, enct_ref, encb_ref,
              cb_ref, z_ref, idx_ref, xp, h1p, p2, hft):
    i = pl.program_id(0)

    @pl.when(i == 0)
    def _zero():
        xp[:] = jnp.zeros((900, BB), jnp.float32)
        h1p[:] = jnp.zeros((7200, BB), jnp.float32)

    for y in range(28):
        xp[y * 30:y * 30 + 28, :] = x_ref[y, :, 0, :]

    a1 = a1_ref[:]
    c1b = c1b_ref[:]
    for i1 in range(14):
        rows = xp[2 * i1 * 30: 2 * i1 * 30 + 90, :]
        val = jnp.maximum(_dot(a1, rows) + c1b, 0.0)
        h1p[i1 * 480: i1 * 480 + 448, :] = val

    for ij in range(NPIX7):
        i2, j2 = divmod(ij, 7)
        for dy in range(3):
            src = ((2 * i2 + dy) * 15 + 2 * j2) * 32
            p2[dy * 96:(dy + 1) * 96, ij * BB:(ij + 1) * BB] = \
                h1p[src: src + 96, :]
    h2 = jnp.maximum(_dot(w2f_ref[:], p2[:]) + c2b_ref[:], 0.0)

    for ij in range(NPIX7):
        hft[ij * 64:(ij + 1) * 64, :] = h2[:, ij * BB:(ij + 1) * BB]
    zt = _dot(enct_ref[:], hft[:]) + encb_ref[:]
    z_ref[:] = zt

    cb = cb_ref[:]
    cbn = jnp.sum(cb * cb, axis=1, keepdims=True)
    dist = cbn - 2.0 * _dot(cb, zt)
    mn = jnp.min(dist, axis=0, keepdims=True)
    ki = jax.lax.broadcasted_iota(jnp.int32, (NUM_EMBED, BB), 0)
    cand = jnp.where(dist <= mn, ki, NUM_EMBED)
    idx = jnp.min(cand, axis=0, keepdims=True)
    idx_ref[:] = idx.reshape(1, 1, BB)


def _shift2d(src, su, sv):
    plan = []
    v0 = max(0, -sv)
    v1 = 7 - max(0, sv)
    for u in range(max(0, -su), 7 - max(0, su)):
        dst = (u * 7 + v0) * BB
        srcl = ((u + su) * 7 + v0 + sv) * BB
        plan.append((dst, srcl, (v1 - v0) * BB))
    return plan


def _dec_body(idx_ref, dct_ref, decb_ref, ws1_ref, b1_ref,
              wsx_ref, b2_ref, w3_ref, b3_ref, perm_ref,
              rec_ref, srcs1, s2, ish, sall3):
    i = pl.program_id(0)

    @pl.when(i == 0)
    def _zero():
        srcs1[:] = jnp.zeros((128, NL), jnp.float32)
        s2[:] = jnp.zeros((384, NL), jnp.float32)
        ish[:] = jnp.zeros((128, NL), jnp.float32)
        sall3[:] = jnp.zeros((1152, NL), jnp.float32)

    ki = jax.lax.broadcasted_iota(jnp.int32, (NUM_EMBED, BB), 0)
    oh = (ki == idx_ref[0]).astype(jnp.float32)
    d1t = jnp.maximum(_dot(dct_ref[:], oh) + decb_ref[:], 0.0)

    for p in range(NPIX7):
        srcs1[0:32, p * BB:(p + 1) * BB] = d1t[p * 32:(p + 1) * 32, :]
    for r, (su, sv) in enumerate([(-1, 0), (0, -1), (-1, -1)]):
        for dst, src, w in _shift2d(None, su, sv):
            srcs1[(r + 1) * 32:(r + 2) * 32, dst:dst + w] = \
                srcs1[0:32, src:src + w]

    p1 = jnp.maximum(_dot(ws1_ref[:], srcs1[:]) + b1_ref[:], 0.0)

    for ry in range(2):
        base = ry * 192
        s2[base:base + 64, :] = p1[ry * 128:ry * 128 + 64, :]
        s2[base + 64:base + 128, :] = p1[ry * 128 + 64:ry * 128 + 128, :]
        for dst, src, w in _shift2d(None, 0, -1):
            s2[base + 128:base + 192, dst:dst + w] = \
                p1[ry * 128 + 64:ry * 128 + 128, src:src + w]
    i0 = _dot(wsx_ref[:], s2[0:192, :])
    i1 = _dot(wsx_ref[:], s2[192:384, :])
    for dst, src, w in _shift2d(None, -1, 0):
        ish[:, dst:dst + w] = i1[0:128, src:src + w]
    r2 = jnp.concatenate([
        i0[256:384, :] + ish[:],
        i0[128:256, :],
        i0[0:128, :] + i1[256:384, :],
        i1[128:256, :],
    ], axis=0)
    r2 = jnp.maximum(r2 + b2_ref[:], 0.0)
    for py in range(4):
        for px in range(4):
            sall3[(py * 6 + px) * 32:(py * 6 + px) * 32 + 32, :] = \
                r2[(py * 4 + px) * 32:(py * 4 + px) * 32 + 32, :]

    for ie in range(6):
        for je in range(6):
            if ie < 4 and je < 4:
                continue
            sy, sx = _E3[ie]['p'], _E3[je]['p']
            su, sv = _E3[ie]['s'], _E3[je]['s']
            srow = (sy * 6 + sx) * 32
            drow = (ie * 6 + je) * 32
            for dst, src, w in _shift2d(None, su, sv):
                sall3[drow:drow + 32, dst:dst + w] = \
                    sall3[srow:srow + 32, src:src + w]

    r3 = jax.nn.sigmoid(_dot(w3_ref[:], sall3[:]) + b3_ref[:])

    cat = [r3[:, g * BB:(g + 1) * BB] for g in range(NPIX7)]
    cat2 = jnp.concatenate(cat, axis=0)
    rec_ref[:] = _dot(perm_ref[:], cat2)


def _loss_body(q_ref, zt_ref, loss_ref):
    i = pl.program_id(0)

    @pl.when(i == 0)
    def _zero():
        loss_ref[:] = jnp.zeros((8, 128), jnp.float32)

    qt = q_ref[:, 0:EMBED_DIM].T
    diff = qt - zt_ref[:]
    loss_ref[:] += jnp.sum(diff * diff)


def _sc_gather(codebook, idx):
    table = jnp.pad(codebook, ((0, 0), (0, 128 - EMBED_DIM)))
    info = plsc.get_sparse_core_info()
    nw = info.num_cores * info.num_subcores
    n = idx.shape[0]
    bpw = n // nw
    mesh = plsc.VectorSubcoreMesh(core_axis_name="c", subcore_axis_name="s")

    @functools.partial(
        pl.kernel, mesh=mesh,
        out_type=jax.ShapeDtypeStruct((n, 128), jnp.float32),
        scratch_types=[
            pltpu.VMEM((bpw,), jnp.int32),
            pltpu.VMEM((bpw, 128), jnp.float32),
            pltpu.SemaphoreType.DMA,
        ],
    )
    def k(table_hbm, idx_hbm, out_hbm, idx_v, rows_v, sem):
        wid = lax.axis_index("s") * info.num_cores + lax.axis_index("c")
        base = wid * bpw
        pltpu.sync_copy(idx_hbm.at[pl.ds(base, bpw)], idx_v)
        pltpu.async_copy(table_hbm.at[idx_v], rows_v, sem).wait()
        pltpu.sync_copy(rows_v, out_hbm.at[pl.ds(base, bpw)])

    return k(table, idx)


def kernel(inputs, conv1_w, conv1_b, conv2_w, conv2_b, enc_w, enc_b, codebook,
           dec_w, dec_b, deconv1_w, deconv1_b, deconv2_w, deconv2_b,
           deconv3_w, deconv3_b):
    batch = inputs.shape[0]
    nblk = batch // BB
    f32 = jnp.float32

    w1p = conv1_w[:, :, 0, :]
    sel = np.zeros((14, 3, 30), np.float32)
    for j in range(14):
        for dx in range(3):
            sel[j, dx, 2 * j + dx] = 1.0
    a1 = jnp.einsum('yxc,jxm->jcym', w1p, jnp.asarray(sel)).reshape(448, 90)
    c1b = jnp.tile(conv1_b.reshape(1, 32), (14, 1)).reshape(448, 1)
    w2f = conv2_w.reshape(288, 64).T
    c2b = conv2_b.reshape(64, 1)
    enct = enc_w.T
    encb = enc_b.reshape(64, 1)

    dct = (codebook @ dec_w).T
    decb = dec_b.reshape(1568, 1)
    feeds1 = [[(0, (2, 2)), (1, (0, 2)), (2, (2, 0)), (3, (0, 0))],
              [(0, (2, 1)), (1, (0, 1))],
              [(0, (1, 2)), (2, (1, 0))],
              [(0, (1, 1))]]
    sel1 = np.zeros((4, 4, 3, 3), np.float32)
    for p, feeds in enumerate(feeds1):
        for (r, (dy, dx)) in feeds:
            sel1[p, r, dy, dx] = 1.0
    ws1 = jnp.einsum('pryx,yxio->pori', jnp.asarray(sel1),
                     deconv1_w).reshape(256, 128)
    b1 = jnp.tile(deconv1_b.reshape(1, 64), (4, 1)).reshape(256, 1)

    selx = np.zeros((3, 4, 3), np.float32)
    for e, ex in enumerate(_E2):
        for (px, tx) in ex['feeds']:
            selx[e, px, tx] = 1.0
    wsx = jnp.einsum('ept,ytio->ypoei', jnp.asarray(selx),
                     deconv2_w).reshape(384, 192)
    b2 = jnp.tile(deconv2_b.reshape(1, 32), (16, 1)).reshape(512, 1)

    sel3 = np.zeros((16, 36, 3, 3), np.float32)
    for ie in range(6):
        for je in range(6):
            s36 = ie * 6 + je
            for (py, ty) in _E3[ie]['feeds']:
                for (px, tx) in _E3[je]['feeds']:
                    sel3[py * 4 + px, s36, ty, tx] = 1.0
    w3 = jnp.einsum('pstu,tuc->psc', jnp.asarray(sel3),
                    deconv3_w[:, :, :, 0]).reshape(16, 1152)
    b3 = deconv3_b.reshape(1, 1)

    perm = np.zeros((784, 784), np.float32)
    for gy in range(7):
        for gx in range(7):
            for py in range(4):
                for px in range(4):
                    perm[(4 * gy + py) * 28 + 4 * gx + px,
                         (gy * 7 + gx) * 16 + py * 4 + px] = 1.0
    perm = jnp.asarray(perm)

    x2d = inputs.transpose(1, 2, 3, 0)

    full = lambda shape: pl.BlockSpec(shape, lambda i: tuple(0 for _ in shape))
    zt_all, idx_all = pl.pallas_call(
        _enc_body,
        grid=(nblk,),
        in_specs=[
            pl.BlockSpec((28, 28, 1, BB), lambda i: (0, 0, 0, i)),
            full((448, 90)), full((448, 1)), full((64, 288)), full((64, 1)),
            full((64, 3136)), full((64, 1)), full((NUM_EMBED, EMBED_DIM)),
        ],
        out_specs=[
            pl.BlockSpec((EMBED_DIM, BB), lambda i: (0, i)),
            pl.BlockSpec((1, 1, BB), lambda i: (i, 0, 0)),
        ],
        out_shape=[
            jax.ShapeDtypeStruct((EMBED_DIM, batch), f32),
            jax.ShapeDtypeStruct((nblk, 1, BB), jnp.int32),
        ],
        scratch_shapes=[
            pltpu.VMEM((900, BB), f32), pltpu.VMEM((7200, BB), f32),
            pltpu.VMEM((288, NL), f32), pltpu.VMEM((3136, BB), f32),
        ],
        compiler_params=pltpu.CompilerParams(
            vmem_limit_bytes=120 * 1024 * 1024),
        interpret=_INTERPRET,
    )(x2d, a1, c1b, w2f, c2b, enct, encb, codebook)

    q = _sc_gather(codebook, idx_all.reshape(batch))

    rec2d = pl.pallas_call(
        _dec_body,
        grid=(nblk,),
        in_specs=[
            pl.BlockSpec((1, 1, BB), lambda i: (i, 0, 0)),
            full((1568, 64)), full((1568, 1)), full((256, 128)), full((256, 1)),
            full((384, 192)), full((512, 1)), full((16, 1152)), full((1, 1)),
            full((784, 784)),
        ],
        out_specs=pl.BlockSpec((784, BB), lambda i: (0, i)),
        out_shape=jax.ShapeDtypeStruct((784, batch), f32),
        scratch_shapes=[
            pltpu.VMEM((128, NL), f32), pltpu.VMEM((384, NL), f32),
            pltpu.VMEM((128, NL), f32), pltpu.VMEM((1152, NL), f32),
        ],
        compiler_params=pltpu.CompilerParams(
            vmem_limit_bytes=120 * 1024 * 1024),
        interpret=_INTERPRET,
    )(idx_all, dct, decb, ws1, b1, wsx, b2, w3, b3, perm)

    lb = min(512, batch)
    loss_acc = pl.pallas_call(
        _loss_body,
        grid=(batch // lb,),
        in_specs=[
            pl.BlockSpec((lb, 128), lambda i: (i, 0)),
            pl.BlockSpec((EMBED_DIM, lb), lambda i: (0, i)),
        ],
        out_specs=pl.BlockSpec((8, 128), lambda i: (0, 0)),
        out_shape=jax.ShapeDtypeStruct((8, 128), f32),
        interpret=_INTERPRET,
    )(q, zt_all)

    vq_loss = 2.0 * loss_acc[0, 0] / (batch * EMBED_DIM)
    reconstructed = jax.lax.reshape(rec2d, (batch, 28, 28, 1),
                                    dimensions=(1, 0))
    return (reconstructed, vq_loss)
